# Initial kernel scaffold; baseline (speedup 1.0000x reference)
import jax
import jax.numpy as jnp
from jax.experimental import pallas as pl


def kernel(node_types, adj, gru_Wih, gru_Whh, gru_bih, gru_bhh, Wg, bg, Wm, W1, b1, W2, b2):
    raise NotImplementedError("write your pallas kernel here")


# speedup vs baseline: 2.8268x; 2.8268x over previous
import jax
import jax.numpy as jnp
from jax.experimental import pallas as pl
from jax.experimental.pallas import tpu as pltpu

B = 256
MAX_N = 64
NVT = 20
HS = 501
NZ = 56
HP = 512
GP = 3 * HP
NP = 128
ZP = 128


def _body(nt_ref, adjT_ref, wih_ref, whh_ref, bhh_ref, wg_ref, wgb_ref,
          wm_ref, wmb_ref, w1_ref, b1_ref, w2_ref, b2_ref,
          mu_ref, lv_ref, m_ref):
    m_ref[...] = jnp.zeros_like(m_ref)

    def step(v, hv_prev):
        del hv_prev
        vmask = (jax.lax.broadcasted_iota(jnp.int32, (1, 1, MAX_N), 2)
                 == v).astype(jnp.float32)
        c = jnp.sum(adjT_ref[...] * vmask, axis=2, keepdims=True)
        h_in = jnp.sum(m_ref[...] * c, axis=0)
        ntv = jnp.sum(nt_ref[...]
                      * (jax.lax.broadcasted_iota(jnp.int32, (1, MAX_N), 1)
                         == v).astype(jnp.int32),
                      axis=1, keepdims=True)
        onehot = (jax.lax.broadcasted_iota(jnp.int32, (B, NP), 1)
                  == ntv).astype(jnp.float32)
        gi = jnp.dot(onehot, wih_ref[...],
                     preferred_element_type=jnp.float32)
        gh = jnp.dot(h_in, whh_ref[...],
                     preferred_element_type=jnp.float32) + bhh_ref[...]
        r = jax.nn.sigmoid(gi[:, 0:HP] + gh[:, 0:HP])
        z = jax.nn.sigmoid(gi[:, HP:2 * HP] + gh[:, HP:2 * HP])
        n = jnp.tanh(gi[:, 2 * HP:GP] + r * gh[:, 2 * HP:GP])
        hv = (1.0 - z) * n + z * h_in
        gate = jax.nn.sigmoid(
            jnp.dot(hv, wg_ref[...], preferred_element_type=jnp.float32)
            + wgb_ref[pl.ds(v, 1), 0, :])
        mapped = (jnp.dot(hv, wm_ref[...], preferred_element_type=jnp.float32)
                  + wmb_ref[pl.ds(v, 1), 0, :])
        m_ref[pl.ds(v, 1)] = (gate * mapped)[None]
        return hv

    hv = jax.lax.fori_loop(0, MAX_N, step,
                           jnp.zeros((B, HP), jnp.float32))
    mu_ref[...] = jnp.dot(hv, w1_ref[...],
                          preferred_element_type=jnp.float32) + b1_ref[...]
    lv_ref[...] = jnp.dot(hv, w2_ref[...],
                          preferred_element_type=jnp.float32) + b2_ref[...]


def _pack3(wt, rows):
    out = jnp.zeros((rows, GP), jnp.float32)
    for k in range(3):
        out = out.at[:wt.shape[0], k * HP:k * HP + HS].set(
            wt[:, k * HS:(k + 1) * HS])
    return out


def kernel(node_types, adj, gru_Wih, gru_Whh, gru_bih, gru_bhh,
           Wg, bg, Wm, W1, b1, W2, b2):
    nt = node_types.astype(jnp.int32)
    adjT = jnp.transpose(adj, (1, 0, 2))

    wih = _pack3(gru_Wih.T + gru_bih[None, :], NP)
    whh = _pack3(gru_Whh.T, HP)
    bhh = _pack3(gru_bhh[None, :], 1)

    wg = jnp.zeros((HP, HP), jnp.float32).at[:HS, :HS].set(Wg[:, :HS].T)
    wgb = jnp.zeros((MAX_N, 1, HP), jnp.float32).at[:, 0, :HS].set(
        bg[None, :] + Wg[:, HS:].T)
    wm = jnp.zeros((HP, HP), jnp.float32).at[:HS, :HS].set(Wm[:, :HS].T)
    wmb = jnp.zeros((MAX_N, 1, HP), jnp.float32).at[:, 0, :HS].set(Wm[:, HS:].T)

    w1 = jnp.zeros((HP, ZP), jnp.float32).at[:HS, :NZ].set(W1.T)
    b1p = jnp.zeros((1, ZP), jnp.float32).at[0, :NZ].set(b1)
    w2 = jnp.zeros((HP, ZP), jnp.float32).at[:HS, :NZ].set(W2.T)
    b2p = jnp.zeros((1, ZP), jnp.float32).at[0, :NZ].set(b2)

    mu, lv = pl.pallas_call(
        _body,
        out_shape=(jax.ShapeDtypeStruct((B, ZP), jnp.float32),
                   jax.ShapeDtypeStruct((B, ZP), jnp.float32)),
        scratch_shapes=[pltpu.VMEM((MAX_N, B, HP), jnp.float32)],
        compiler_params=pltpu.CompilerParams(
            vmem_limit_bytes=120 * 1024 * 1024),
    )(nt, adjT, wih, whh, bhh, wg, wgb, wm, wmb, w1, b1p, w2, b2p)
    return (mu[:, :NZ], lv[:, :NZ])


# speedup vs baseline: 3.7324x; 1.3204x over previous
import jax
import jax.numpy as jnp
from jax.experimental import pallas as pl
from jax.experimental.pallas import tpu as pltpu

B = 256
MAX_N = 64
NVT = 20
HS = 501
NZ = 56
HP = 512
GP = 3 * HP
NP = 128
ZP = 128
PHASES = 8
PLEN = MAX_N // PHASES


def _body(nt_ref, adjT_ref, wih_ref, whh_ref, bhh_ref, wg_ref, wgb_ref,
          wm_ref, wmb_ref, w1_ref, b1_ref, w2_ref, b2_ref,
          mu_ref, lv_ref, m_ref):
    m_ref[...] = jnp.zeros_like(m_ref)

    def make_step(pref):
        def step(v, hv_prev):
            del hv_prev
            vmask = (jax.lax.broadcasted_iota(jnp.int32, (1, 1, MAX_N), 2)
                     == v).astype(jnp.bfloat16)
            c = jnp.sum(adjT_ref[0:pref] * vmask, axis=2,
                        keepdims=True)
            h_in = jnp.sum(m_ref[0:pref] * c, axis=0,
                           dtype=jnp.float32)
            ntv = jnp.sum(nt_ref[...]
                          * (jax.lax.broadcasted_iota(jnp.int32, (1, MAX_N), 1)
                             == v).astype(jnp.int32),
                          axis=1, keepdims=True)
            onehot = (jax.lax.broadcasted_iota(jnp.int32, (B, NP), 1)
                      == ntv).astype(jnp.bfloat16)
            gi = jnp.dot(onehot, wih_ref[...],
                         preferred_element_type=jnp.float32)
            gh = jnp.dot(h_in.astype(jnp.bfloat16), whh_ref[...],
                         preferred_element_type=jnp.float32) + bhh_ref[...]
            r = jax.nn.sigmoid(gi[:, 0:HP] + gh[:, 0:HP])
            z = jax.nn.sigmoid(gi[:, HP:2 * HP] + gh[:, HP:2 * HP])
            n = jnp.tanh(gi[:, 2 * HP:GP] + r * gh[:, 2 * HP:GP])
            hv = (1.0 - z) * n + z * h_in
            hvb = hv.astype(jnp.bfloat16)
            gate = jax.nn.sigmoid(
                jnp.dot(hvb, wg_ref[...], preferred_element_type=jnp.float32)
                + wgb_ref[pl.ds(v, 1), 0, :])
            mapped = (jnp.dot(hvb, wm_ref[...],
                              preferred_element_type=jnp.float32)
                      + wmb_ref[pl.ds(v, 1), 0, :])
            m_ref[pl.ds(v, 1)] = (gate * mapped).astype(jnp.bfloat16)[None]
            return hv
        return step

    hv = jnp.zeros((B, HP), jnp.float32)
    for p in range(PHASES):
        hv = jax.lax.fori_loop(p * PLEN, (p + 1) * PLEN,
                               make_step((p + 1) * PLEN), hv)
    mu_ref[...] = jnp.dot(hv, w1_ref[...],
                          preferred_element_type=jnp.float32) + b1_ref[...]
    lv_ref[...] = jnp.dot(hv, w2_ref[...],
                          preferred_element_type=jnp.float32) + b2_ref[...]


def _pack3(wt, rows, dtype):
    out = jnp.zeros((rows, GP), jnp.float32)
    for k in range(3):
        out = out.at[:wt.shape[0], k * HP:k * HP + HS].set(
            wt[:, k * HS:(k + 1) * HS])
    return out.astype(dtype)


def kernel(node_types, adj, gru_Wih, gru_Whh, gru_bih, gru_bhh,
           Wg, bg, Wm, W1, b1, W2, b2):
    f32, bf16 = jnp.float32, jnp.bfloat16
    nt = node_types.astype(jnp.int32)
    adjT = jnp.transpose(adj, (1, 0, 2)).astype(bf16)

    wih = _pack3(gru_Wih.T + gru_bih[None, :], NP, bf16)
    whh = _pack3(gru_Whh.T, HP, bf16)
    bhh = _pack3(gru_bhh[None, :], 1, f32)

    wg = jnp.zeros((HP, HP), f32).at[:HS, :HS].set(Wg[:, :HS].T).astype(bf16)
    wgb = jnp.zeros((MAX_N, 1, HP), f32).at[:, 0, :HS].set(
        bg[None, :] + Wg[:, HS:].T)
    wm = jnp.zeros((HP, HP), f32).at[:HS, :HS].set(Wm[:, :HS].T).astype(bf16)
    wmb = jnp.zeros((MAX_N, 1, HP), f32).at[:, 0, :HS].set(Wm[:, HS:].T)

    w1 = jnp.zeros((HP, ZP), f32).at[:HS, :NZ].set(W1.T)
    b1p = jnp.zeros((1, ZP), f32).at[0, :NZ].set(b1)
    w2 = jnp.zeros((HP, ZP), f32).at[:HS, :NZ].set(W2.T)
    b2p = jnp.zeros((1, ZP), f32).at[0, :NZ].set(b2)

    mu, lv = pl.pallas_call(
        _body,
        out_shape=(jax.ShapeDtypeStruct((B, ZP), f32),
                   jax.ShapeDtypeStruct((B, ZP), f32)),
        scratch_shapes=[pltpu.VMEM((MAX_N, B, HP), bf16)],
        compiler_params=pltpu.CompilerParams(
            vmem_limit_bytes=120 * 1024 * 1024),
    )(nt, adjT, wih, whh, bhh, wg, wgb, wm, wmb, w1, b1p, w2, b2p)
    return (mu[:, :NZ], lv[:, :NZ])


# speedup vs baseline: 4.1822x; 1.1205x over previous
import jax
import jax.numpy as jnp
from jax.experimental import pallas as pl
from jax.experimental.pallas import tpu as pltpu

B = 256
MAX_N = 64
NVT = 20
HS = 501
NZ = 56
HP = 512
GP = 3 * HP
NP = 128
ZP = 128
PHASES = 8
PLEN = MAX_N // PHASES


def _body(nt_ref, adjP_ref, wih_ref, whh_ref, bhh_ref, wg_ref, wgb_ref,
          wm_ref, wmb_ref, w1_ref, b1_ref, w2_ref, b2_ref,
          mu_ref, lv_ref, m_ref):
    m_ref[...] = jnp.zeros_like(m_ref)
    vlane = jax.lax.broadcasted_iota(jnp.int32, (1, MAX_N), 1)

    def make_step(pref):
        def step(v, hv_prev):
            del hv_prev
            c = adjP_ref[pl.ds(v, 1)][0, 0:pref]
            prod = m_ref[0:pref] * c
            pairs = (prod.reshape(pref // 2, 2, HP, B)[:, 0]
                     + prod.reshape(pref // 2, 2, HP, B)[:, 1])
            h_in = jnp.sum(pairs, axis=0, dtype=jnp.float32)
            ntv = nt_ref[pl.ds(v, 1)][0]
            onehot = (jax.lax.broadcasted_iota(jnp.int32, (NP, B), 0)
                      == ntv).astype(jnp.bfloat16)
            gi = jnp.dot(wih_ref[...], onehot,
                         preferred_element_type=jnp.float32)
            gh = jnp.dot(whh_ref[...], h_in.astype(jnp.bfloat16),
                         preferred_element_type=jnp.float32) + bhh_ref[...]
            r = jax.nn.sigmoid(gi[0:HP] + gh[0:HP])
            z = jax.nn.sigmoid(gi[HP:2 * HP] + gh[HP:2 * HP])
            n = jnp.tanh(gi[2 * HP:GP] + r * gh[2 * HP:GP])
            hv = (1.0 - z) * n + z * h_in
            vmask = (vlane == v).astype(jnp.float32)
            gbias = jnp.sum(wgb_ref[...] * vmask, axis=1, keepdims=True)
            mbias = jnp.sum(wmb_ref[...] * vmask, axis=1, keepdims=True)
            hvb = hv.astype(jnp.bfloat16)
            gate = jax.nn.sigmoid(
                jnp.dot(wg_ref[...], hvb,
                        preferred_element_type=jnp.float32) + gbias)
            mapped = (jnp.dot(wm_ref[...], hvb,
                              preferred_element_type=jnp.float32) + mbias)
            m_ref[pl.ds(v, 1)] = (gate * mapped).astype(jnp.bfloat16)[None]
            return hv
        return step

    hv = jnp.zeros((HP, B), jnp.float32)
    for p in range(PHASES):
        hv = jax.lax.fori_loop(p * PLEN, (p + 1) * PLEN,
                               make_step((p + 1) * PLEN), hv)
    mu_ref[...] = jnp.dot(w1_ref[...], hv,
                          preferred_element_type=jnp.float32) + b1_ref[...]
    lv_ref[...] = jnp.dot(w2_ref[...], hv,
                          preferred_element_type=jnp.float32) + b2_ref[...]


def _pack3(w, cols, dtype):
    out = jnp.zeros((GP, cols), jnp.float32)
    for k in range(3):
        out = out.at[k * HP:k * HP + HS, :w.shape[1]].set(
            w[k * HS:(k + 1) * HS, :])
    return out.astype(dtype)


def kernel(node_types, adj, gru_Wih, gru_Whh, gru_bih, gru_bhh,
           Wg, bg, Wm, W1, b1, W2, b2):
    f32, bf16 = jnp.float32, jnp.bfloat16
    nt = node_types.astype(jnp.int32).T[:, None, :]
    adjP = jnp.transpose(adj, (2, 1, 0))[:, :, None, :].astype(bf16)

    wih = _pack3(gru_Wih + gru_bih[:, None], NP, bf16)
    whh = _pack3(gru_Whh, HP, bf16)
    bhh = _pack3(gru_bhh[:, None], 1, f32)

    wg = jnp.zeros((HP, HP), f32).at[:HS, :HS].set(Wg[:, :HS]).astype(bf16)
    wgb = jnp.zeros((HP, MAX_N), f32).at[:HS, :].set(
        bg[:, None] + Wg[:, HS:])
    wm = jnp.zeros((HP, HP), f32).at[:HS, :HS].set(Wm[:, :HS]).astype(bf16)
    wmb = jnp.zeros((HP, MAX_N), f32).at[:HS, :].set(Wm[:, HS:])

    w1 = jnp.zeros((ZP, HP), f32).at[:NZ, :HS].set(W1)
    b1p = jnp.zeros((ZP, 1), f32).at[:NZ, 0].set(b1)
    w2 = jnp.zeros((ZP, HP), f32).at[:NZ, :HS].set(W2)
    b2p = jnp.zeros((ZP, 1), f32).at[:NZ, 0].set(b2)

    mu, lv = pl.pallas_call(
        _body,
        out_shape=(jax.ShapeDtypeStruct((ZP, B), f32),
                   jax.ShapeDtypeStruct((ZP, B), f32)),
        scratch_shapes=[pltpu.VMEM((MAX_N, HP, B), bf16)],
        compiler_params=pltpu.CompilerParams(
            vmem_limit_bytes=120 * 1024 * 1024),
    )(nt, adjP, wih, whh, bhh, wg, wgb, wm, wmb, w1, b1p, w2, b2p)
    return (mu.T[:, :NZ], lv.T[:, :NZ])
